# Initial kernel scaffold; baseline (speedup 1.0000x reference)
import jax
import jax.numpy as jnp
from jax.experimental import pallas as pl


def kernel(head_ids, relation_ids, tail_ids, entity_table, relation_table):
    raise NotImplementedError("write your pallas kernel here")


# speedup vs baseline: 2.8710x; 2.8710x over previous
import functools

import jax
import jax.numpy as jnp
from jax import lax
from jax.experimental import pallas as pl
from jax.experimental.pallas import tpu as pltpu
from jax.experimental.pallas import tpu_sc as plsc

_B = 16384
_W = 128
_NW = 32
_RPW = _B // _NW
_CHUNK = 128
_NCHUNK = _RPW // _CHUNK


def _sc_body(hid_hbm, rid_hbm, tid_hbm, ent_hbm, rel_hbm, out_hbm,
             hid_v, rid_v, tid_v, hbuf, rbuf, tbuf, sbuf, outv, sem):
    c = lax.axis_index("c")
    s = lax.axis_index("s")
    wid = s * 2 + c
    base = wid * _RPW

    for j in range(_NCHUNK):
        off = base + j * _CHUNK
        pltpu.sync_copy(hid_hbm.at[pl.ds(off, _CHUNK)], hid_v.at[j])
        pltpu.sync_copy(rid_hbm.at[pl.ds(off, _CHUNK)], rid_v.at[j])
        pltpu.sync_copy(tid_hbm.at[pl.ds(off, _CHUNK)], tid_v.at[j])

    row_iota = lax.iota(jnp.int32, 16)

    for j in range(_NCHUNK):
        cph = pltpu.async_copy(ent_hbm.at[hid_v.at[j]], hbuf, sem)
        cpr = pltpu.async_copy(rel_hbm.at[rid_v.at[j]], rbuf, sem)
        cpt = pltpu.async_copy(ent_hbm.at[tid_v.at[j]], tbuf, sem)
        cph.wait()
        cpr.wait()
        cpt.wait()

        def group(g, _, j=j):
            def row(r, scores):
                rr = g * 16 + r
                acc = jnp.zeros((16,), jnp.float32)
                for k in range(4):
                    sl_re = pl.ds(k * 16, 16)
                    sl_im = pl.ds(64 + k * 16, 16)
                    hre = hbuf[rr, sl_re]
                    him = hbuf[rr, sl_im]
                    rre = rbuf[rr, sl_re]
                    rim = rbuf[rr, sl_im]
                    tre = tbuf[rr, sl_re]
                    tim = tbuf[rr, sl_im]
                    m1 = rre * tre + rim * tim
                    m2 = rre * tim - rim * tre
                    acc = acc + hre * m1 + him * m2
                srow = jnp.sum(acc)
                return jnp.where(row_iota == r, srow, scores)

            scores = lax.fori_loop(0, 16, row, jnp.zeros((16,), jnp.float32))
            outv[pl.ds(j * _CHUNK + g * 16, 16)] = scores
            return 0

        lax.fori_loop(0, _CHUNK // 16, group, 0)

    pltpu.sync_copy(outv, out_hbm.at[pl.ds(base, _RPW)])


@functools.partial(
    pl.kernel,
    out_type=jax.ShapeDtypeStruct((_B,), jnp.float32),
    mesh=plsc.VectorSubcoreMesh(core_axis_name="c", subcore_axis_name="s"),
    scratch_types=[
        pltpu.VMEM((_NCHUNK, _CHUNK), jnp.int32),
        pltpu.VMEM((_NCHUNK, _CHUNK), jnp.int32),
        pltpu.VMEM((_NCHUNK, _CHUNK), jnp.int32),
        pltpu.VMEM((_CHUNK, _W), jnp.float32),
        pltpu.VMEM((_CHUNK, _W), jnp.float32),
        pltpu.VMEM((_CHUNK, _W), jnp.float32),
        pltpu.VMEM((16, 16), jnp.float32),
        pltpu.VMEM((_RPW,), jnp.float32),
        pltpu.SemaphoreType.DMA,
    ],
    compiler_params=pltpu.CompilerParams(needs_layout_passes=False),
)
def _complex_score(hid, rid, tid, ent, rel, out, *scratch):
    _sc_body(hid, rid, tid, ent, rel, out, *scratch)


def kernel(head_ids, relation_ids, tail_ids, entity_table, relation_table):
    return _complex_score(head_ids, relation_ids, tail_ids,
                          entity_table, relation_table)
